# baseline (speedup 1.0000x reference)
import functools

import jax
import jax.numpy as jnp
from jax import lax
from jax.experimental import pallas as pl
from jax.experimental.pallas import tpu as pltpu
from jax.experimental.pallas import tpu_sc as plsc

_N_FIELDS = 26
_VOCAB = 100000
_EMB = 50
_B = 16384
_CONCAT = _N_FIELDS * _EMB
_H = 200
_OUT = 2

_NW = 32
_CHUNK = 4096


def _sc_gather_t(t2, xT):
    mesh = plsc.VectorSubcoreMesh(core_axis_name="c", subcore_axis_name="s")

    @functools.partial(
        pl.kernel,
        out_type=jax.ShapeDtypeStruct((_CONCAT, _B), jnp.float32),
        mesh=mesh,
        scratch_types=[
            pltpu.VMEM((_VOCAB,), jnp.float32),
            pltpu.VMEM((_B,), jnp.int32),
            pltpu.VMEM((_CHUNK,), jnp.float32),
            pltpu.VMEM((_CHUNK,), jnp.float32),
            pltpu.SemaphoreType.DMA,
            pltpu.SemaphoreType.DMA,
        ],
        compiler_params=pltpu.CompilerParams(needs_layout_passes=False),
    )
    def k(t2_hbm, xT_hbm, out_hbm, row_v, idx_v, out0_v, out1_v, sem0, sem1):
        wid = lax.axis_index("s") * 2 + lax.axis_index("c")
        for f in range(_N_FIELDS):
            pltpu.sync_copy(xT_hbm.at[f, :], idx_v)
            off = (-50 * f) % _NW
            e0 = lax.rem(wid + off, _NW)

            @pl.loop(e0, _EMB, step=_NW)
            def _(e, f=f):
                pltpu.sync_copy(t2_hbm.at[f, e, :], row_v)
                r = f * _EMB + e
                bufs = (out0_v, out1_v)
                sems = (sem0, sem1)
                pending = [None, None]
                for ci in range(_B // _CHUNK):
                    ob, sem = bufs[ci % 2], sems[ci % 2]
                    if pending[ci % 2] is not None:
                        pending[ci % 2].wait()
                    base = ci * _CHUNK

                    @plsc.parallel_loop(0, _CHUNK, 16, unroll=8)
                    def _(j, base=base, ob=ob):
                        iv = idx_v[pl.ds(base + j, 16)]
                        ob[pl.ds(j, 16)] = plsc.load_gather(row_v, [iv])

                    pending[ci % 2] = pltpu.async_copy(
                        ob, out_hbm.at[r, pl.ds(base, _CHUNK)], sem)
                for p in pending:
                    if p is not None:
                        p.wait()

    return k(t2, xT)


def _mlp_body(embT_ref, w1_ref, b1_ref, w2_ref, b2_ref, w3_ref, b3_ref,
              w4_ref, b4_ref, w5_ref, b5_ref, o_ref):
    def sig(z):
        return 1.0 / (1.0 + jnp.exp(-z))

    dn = (((0,), (0,)), ((), ()))

    eT = embT_ref[...]
    z = lax.dot_general(w1_ref[...], eT, dn,
                        preferred_element_type=jnp.float32) + b1_ref[...]
    h = sig(z)
    z = lax.dot_general(w2_ref[...], h, dn,
                        preferred_element_type=jnp.float32) + b2_ref[...]
    h = sig(z)
    z = lax.dot_general(w3_ref[...], h, dn,
                        preferred_element_type=jnp.float32) + b3_ref[...]
    h = sig(z)
    z = lax.dot_general(w4_ref[...], h, dn,
                        preferred_element_type=jnp.float32) + b4_ref[...]
    h = sig(z)
    logits = lax.dot_general(w5_ref[...], h, dn,
                             preferred_element_type=jnp.float32) + b5_ref[...]
    m = jnp.max(logits, axis=0, keepdims=True)
    e = jnp.exp(logits - m)
    o_ref[...] = e / jnp.sum(e, axis=0, keepdims=True)


def _tc_mlp_t(embT, W1, b1, W2, b2, W3, b3, W4, b4, W5, b5):
    bt = 2048
    nb = _B // bt
    full = lambda i: (0, 0)
    return pl.pallas_call(
        _mlp_body,
        grid=(nb,),
        in_specs=[
            pl.BlockSpec((_CONCAT, bt), lambda i: (0, i)),
            pl.BlockSpec((_CONCAT, _H), full),
            pl.BlockSpec((_H, 1), full),
            pl.BlockSpec((_H, _H), full),
            pl.BlockSpec((_H, 1), full),
            pl.BlockSpec((_H, _H), full),
            pl.BlockSpec((_H, 1), full),
            pl.BlockSpec((_H, _H), full),
            pl.BlockSpec((_H, 1), full),
            pl.BlockSpec((_H, _OUT), full),
            pl.BlockSpec((_OUT, 1), full),
        ],
        out_specs=pl.BlockSpec((_OUT, bt), lambda i: (0, i)),
        out_shape=jax.ShapeDtypeStruct((_OUT, _B), jnp.float32),
    )(embT, W1, b1.reshape(_H, 1), W2, b2.reshape(_H, 1), W3, b3.reshape(_H, 1),
      W4, b4.reshape(_H, 1), W5, b5.reshape(_OUT, 1))


def kernel(x, tables, W1, b1, W2, b2, W3, b3, W4, b4, W5, b5):
    t2 = jnp.transpose(tables, (0, 2, 1))
    xT = jnp.transpose(x, (1, 0))
    embT = _sc_gather_t(t2, xT)
    probsT = _tc_mlp_t(embT, W1, b1, W2, b2, W3, b3, W4, b4, W5, b5)
    return jnp.transpose(probsT, (1, 0))


# speedup vs baseline: 1.1610x; 1.1610x over previous
import functools

import jax
import jax.numpy as jnp
from jax import lax
from jax.experimental import pallas as pl
from jax.experimental.pallas import tpu as pltpu
from jax.experimental.pallas import tpu_sc as plsc

_N_FIELDS = 26
_VOCAB = 100000
_EMB = 50
_B = 16384
_CONCAT = _N_FIELDS * _EMB
_H = 200
_OUT = 2

_NW = 32
_CHUNK = 4096


def _sc_gather_t(t2, xT):
    mesh = plsc.VectorSubcoreMesh(core_axis_name="c", subcore_axis_name="s")

    @functools.partial(
        pl.kernel,
        out_type=jax.ShapeDtypeStruct((_CONCAT, _B), jnp.float32),
        mesh=mesh,
        scratch_types=[
            pltpu.VMEM((_VOCAB,), jnp.float32),
            pltpu.VMEM((_B,), jnp.int32),
            pltpu.VMEM((_CHUNK,), jnp.float32),
            pltpu.VMEM((_CHUNK,), jnp.float32),
            pltpu.SemaphoreType.DMA,
            pltpu.SemaphoreType.DMA,
        ],
        compiler_params=pltpu.CompilerParams(needs_layout_passes=False),
    )
    def k(t2_hbm, xT_hbm, out_hbm, row_v, idx_v, out0_v, out1_v, sem0, sem1):
        wid = lax.axis_index("s") * 2 + lax.axis_index("c")
        start = 40 * wid + jnp.minimum(wid, 20)
        end = start + jnp.where(wid < 20, 41, 40)
        f_lo = start // _EMB
        f_hi = (end - 1) // _EMB

        @pl.loop(f_lo, f_hi + 1)
        def _(f):
            pltpu.sync_copy(xT_hbm.at[f, :], idx_v)
            r_lo = jnp.maximum(start, _EMB * f)
            r_hi = jnp.minimum(end, _EMB * (f + 1))

            @pl.loop(r_lo, r_hi)
            def _(r, f=f):
                e = r - _EMB * f
                pltpu.sync_copy(t2_hbm.at[f, e, :], row_v)
                bufs = (out0_v, out1_v)
                sems = (sem0, sem1)
                pending = [None, None]
                for ci in range(_B // _CHUNK):
                    ob, sem = bufs[ci % 2], sems[ci % 2]
                    if pending[ci % 2] is not None:
                        pending[ci % 2].wait()
                    base = ci * _CHUNK

                    @plsc.parallel_loop(0, _CHUNK, 16, unroll=8)
                    def _(j, base=base, ob=ob):
                        iv = idx_v[pl.ds(base + j, 16)]
                        ob[pl.ds(j, 16)] = plsc.load_gather(row_v, [iv])

                    pending[ci % 2] = pltpu.async_copy(
                        ob, out_hbm.at[r, pl.ds(base, _CHUNK)], sem)
                for p in pending:
                    if p is not None:
                        p.wait()

    return k(t2, xT)


def _mlp_body(embT_ref, w1_ref, b1_ref, w2_ref, b2_ref, w3_ref, b3_ref,
              w4_ref, b4_ref, w5_ref, b5_ref, o_ref):
    def sig(z):
        return 1.0 / (1.0 + jnp.exp(-z))

    dn = (((0,), (0,)), ((), ()))

    eT = embT_ref[...]
    z = lax.dot_general(w1_ref[...], eT, dn,
                        preferred_element_type=jnp.float32) + b1_ref[...]
    h = sig(z)
    z = lax.dot_general(w2_ref[...], h, dn,
                        preferred_element_type=jnp.float32) + b2_ref[...]
    h = sig(z)
    z = lax.dot_general(w3_ref[...], h, dn,
                        preferred_element_type=jnp.float32) + b3_ref[...]
    h = sig(z)
    z = lax.dot_general(w4_ref[...], h, dn,
                        preferred_element_type=jnp.float32) + b4_ref[...]
    h = sig(z)
    logits = lax.dot_general(w5_ref[...], h, dn,
                             preferred_element_type=jnp.float32) + b5_ref[...]
    m = jnp.max(logits, axis=0, keepdims=True)
    e = jnp.exp(logits - m)
    o_ref[...] = e / jnp.sum(e, axis=0, keepdims=True)


def _tc_mlp_t(embT, W1, b1, W2, b2, W3, b3, W4, b4, W5, b5):
    bt = 2048
    nb = _B // bt
    full = lambda i: (0, 0)
    return pl.pallas_call(
        _mlp_body,
        grid=(nb,),
        in_specs=[
            pl.BlockSpec((_CONCAT, bt), lambda i: (0, i)),
            pl.BlockSpec((_CONCAT, _H), full),
            pl.BlockSpec((_H, 1), full),
            pl.BlockSpec((_H, _H), full),
            pl.BlockSpec((_H, 1), full),
            pl.BlockSpec((_H, _H), full),
            pl.BlockSpec((_H, 1), full),
            pl.BlockSpec((_H, _H), full),
            pl.BlockSpec((_H, 1), full),
            pl.BlockSpec((_H, _OUT), full),
            pl.BlockSpec((_OUT, 1), full),
        ],
        out_specs=pl.BlockSpec((_OUT, bt), lambda i: (0, i)),
        out_shape=jax.ShapeDtypeStruct((_OUT, _B), jnp.float32),
    )(embT, W1, b1.reshape(_H, 1), W2, b2.reshape(_H, 1), W3, b3.reshape(_H, 1),
      W4, b4.reshape(_H, 1), W5, b5.reshape(_OUT, 1))


def kernel(x, tables, W1, b1, W2, b2, W3, b3, W4, b4, W5, b5):
    t2 = jnp.transpose(tables, (0, 2, 1))
    xT = jnp.transpose(x, (1, 0))
    embT = _sc_gather_t(t2, xT)
    probsT = _tc_mlp_t(embT, W1, b1, W2, b2, W3, b3, W4, b4, W5, b5)
    return jnp.transpose(probsT, (1, 0))
